# baseline (speedup 1.0000x reference)
import functools

import jax
import jax.numpy as jnp
from jax import lax
from jax.experimental import pallas as pl
from jax.experimental.pallas import tpu as pltpu
from jax.experimental.pallas import tpu_sc as plsc

F = 26
V = 100000
D = 16
B = 16384

NC = 2
NS = 16
NW = NC * NS
LANES = 16

RPW = B * F // NW
NCHUNK = 8
RPC = RPW // NCHUNK
GSZ = 128
GPC = RPC // GSZ


def _sc_gather(idx_flat, tab_flat):
    mesh = plsc.VectorSubcoreMesh(core_axis_name="c", subcore_axis_name="s")

    @functools.partial(
        pl.kernel,
        mesh=mesh,
        out_type=jax.ShapeDtypeStruct((B * F, D), jnp.float32),
        compiler_params=pltpu.CompilerParams(use_tc_tiling_on_sc=False),
        scratch_types=[
            pltpu.VMEM((RPW,), jnp.int32),
            pltpu.VMEM((RPC, D), jnp.float32),
            pltpu.SemaphoreType.DMA,
        ],
    )
    def k(idx_hbm, tab_hbm, out_hbm, idx_v, rows_v, gsem):
        wid = lax.axis_index("s") * NC + lax.axis_index("c")
        base = wid * RPW

        pltpu.sync_copy(idx_hbm.at[pl.ds(base, RPW)], idx_v)

        iota = lax.iota(jnp.int32, LANES)

        def fix(j, _):
            off = lax.rem(j * LANES + iota, F) * V
            sl = pl.ds(pl.multiple_of(j * LANES, 8), LANES)
            idx_v[sl] = idx_v[sl] + off
            return 0

        lax.fori_loop(0, RPW // LANES, fix, 0)

        def chunk(c, _):
            cbase = pl.multiple_of(c * RPC, 8)
            waits = []
            for j in range(GPC):
                waits.append(
                    pltpu.async_copy(
                        tab_hbm.at[idx_v.at[pl.ds(cbase + j * GSZ, GSZ)]],
                        rows_v.at[pl.ds(j * GSZ, GSZ)],
                        gsem,
                    )
                )
            for w in waits:
                w.wait()
            pltpu.sync_copy(rows_v, out_hbm.at[pl.ds(base + cbase, RPC)])
            return 0

        lax.fori_loop(0, NCHUNK, chunk, 0)

    return k(idx_flat, tab_flat)


def kernel(indices, tables):
    out = _sc_gather(indices.reshape(B * F), tables.reshape(F * V, D))
    return out.reshape(B, F * D)


# speedup vs baseline: 5.0399x; 5.0399x over previous
import functools

import jax
import jax.numpy as jnp
from jax import lax
from jax.experimental import pallas as pl
from jax.experimental.pallas import tpu as pltpu
from jax.experimental.pallas import tpu_sc as plsc

F = 26
V = 100000
D = 16
B = 16384

NC = 2
NS = 16
NW = NC * NS
LANES = 16

R = F * D
RPW = R // NW
OCH = 4096
NOC = B // OCH


def _sc_rowgather(idx_t, tab_t):
    mesh = plsc.VectorSubcoreMesh(core_axis_name="c", subcore_axis_name="s")

    @functools.partial(
        pl.kernel,
        mesh=mesh,
        compiler_params=pltpu.CompilerParams(
            use_tc_tiling_on_sc=True, needs_layout_passes=False
        ),
        out_type=jax.ShapeDtypeStruct((R, B), jnp.float32),
        scratch_types=[
            pltpu.VMEM((V,), jnp.float32),
            pltpu.VMEM((B,), jnp.int32),
            pltpu.VMEM((2, OCH), jnp.float32),
            pltpu.SemaphoreType.DMA,
        ],
    )
    def k(idx_hbm, tab_hbm, out_hbm, row_v, idx_v, obuf, osem):
        wid = lax.axis_index("s") * NC + lax.axis_index("c")
        base = wid * RPW

        def row_body(r, _):
            fld = lax.div(r, D)
            pltpu.sync_copy(idx_hbm.at[fld], idx_v)
            pltpu.sync_copy(tab_hbm.at[r], row_v)

            waits = []
            for c in range(NOC):
                bsel = c % 2
                if len(waits) >= 2:
                    waits.pop(0).wait()

                def jloop(j, _):
                    sl = pl.ds(pl.multiple_of(c * OCH + j * LANES, 8), LANES)
                    iv = idx_v[sl]
                    vals = plsc.load_gather(row_v, [iv])
                    obuf[bsel, pl.ds(pl.multiple_of(j * LANES, 8), LANES)] = vals
                    return 0

                lax.fori_loop(0, OCH // LANES, jloop, 0)
                waits.append(
                    pltpu.async_copy(
                        obuf.at[bsel], out_hbm.at[r, pl.ds(c * OCH, OCH)], osem
                    )
                )
            for w in waits:
                w.wait()
            return 0

        lax.fori_loop(base, base + RPW, row_body, 0)

    return k(idx_t, tab_t)


def kernel(indices, tables):
    idx_t = indices.T
    tab_t = jnp.transpose(tables, (0, 2, 1)).reshape(R, V)
    out_t = _sc_rowgather(idx_t, tab_t)
    return out_t.T.reshape(B, F * D)


# speedup vs baseline: 8.8568x; 1.7573x over previous
import functools

import jax
import jax.numpy as jnp
from jax import lax
from jax.experimental import pallas as pl
from jax.experimental.pallas import tpu as pltpu
from jax.experimental.pallas import tpu_sc as plsc

F = 26
V = 100000
D = 16
B = 16384

NC = 2
NS = 16
NW = NC * NS
LANES = 16

R = F * D
RPW = R // NW
OCH = 4096
NOC = B // OCH


def _sc_rowgather(idx_t, tab_t):
    mesh = plsc.VectorSubcoreMesh(core_axis_name="c", subcore_axis_name="s")

    @functools.partial(
        pl.kernel,
        mesh=mesh,
        compiler_params=pltpu.CompilerParams(
            use_tc_tiling_on_sc=True, needs_layout_passes=False
        ),
        out_type=jax.ShapeDtypeStruct((R, B), jnp.float32),
        scratch_types=[
            pltpu.VMEM((V,), jnp.float32),
            pltpu.VMEM((B,), jnp.int32),
            pltpu.VMEM((2, OCH), jnp.float32),
            pltpu.SemaphoreType.DMA,
        ],
    )
    def k(idx_hbm, tab_hbm, out_hbm, row_v, idx_v, obuf, osem):
        wid = lax.axis_index("s") * NC + lax.axis_index("c")
        base = wid * RPW

        def row_body(r, _):
            fld = lax.div(r, D)
            pltpu.sync_copy(idx_hbm.at[fld], idx_v)
            pltpu.sync_copy(tab_hbm.at[r], row_v)

            waits = []
            for c in range(NOC):
                bsel = c % 2
                if len(waits) >= 2:
                    waits.pop(0).wait()

                @plsc.parallel_loop(0, OCH // LANES, unroll=8)
                def _(j):
                    sl = pl.ds(pl.multiple_of(c * OCH + j * LANES, 8), LANES)
                    iv = idx_v[sl]
                    vals = plsc.load_gather(row_v, [iv])
                    obuf[bsel, pl.ds(pl.multiple_of(j * LANES, 8), LANES)] = vals
                waits.append(
                    pltpu.async_copy(
                        obuf.at[bsel], out_hbm.at[r, pl.ds(c * OCH, OCH)], osem
                    )
                )
            for w in waits:
                w.wait()
            return 0

        lax.fori_loop(base, base + RPW, row_body, 0)

    return k(idx_t, tab_t)


def kernel(indices, tables):
    idx_t = indices.T
    tab_t = jnp.transpose(tables, (0, 2, 1)).reshape(R, V)
    out_t = _sc_rowgather(idx_t, tab_t)
    return out_t.T.reshape(B, F * D)


# speedup vs baseline: 9.5179x; 1.0746x over previous
import functools

import jax
import jax.numpy as jnp
from jax import lax
from jax.experimental import pallas as pl
from jax.experimental.pallas import tpu as pltpu
from jax.experimental.pallas import tpu_sc as plsc

F = 26
V = 100000
D = 16
B = 16384

NC = 2
NS = 16
NW = NC * NS
LANES = 16

R = F * D
RPW = R // NW
OCH = 4096
NOC = B // OCH


def _sc_rowgather(idx_t, tab_t):
    mesh = plsc.VectorSubcoreMesh(core_axis_name="c", subcore_axis_name="s")

    @functools.partial(
        pl.kernel,
        mesh=mesh,
        compiler_params=pltpu.CompilerParams(
            use_tc_tiling_on_sc=True, needs_layout_passes=False
        ),
        out_type=jax.ShapeDtypeStruct((R, B), jnp.float32),
        scratch_types=[
            pltpu.VMEM((V,), jnp.float32),
            pltpu.VMEM((B,), jnp.int32),
            pltpu.VMEM((2, OCH), jnp.float32),
            pltpu.SemaphoreType.DMA,
            pltpu.SemaphoreType.DMA,
        ],
    )
    def k(idx_hbm, tab_hbm, out_hbm, row_v, idx_v, obuf, osem, rsem):
        wid = lax.axis_index("s") * NC + lax.axis_index("c")
        base = wid * RPW

        waits = []
        prev_fld = jnp.int32(-1)
        for k_row in range(RPW):
            r = base + k_row
            fld = lax.div(r, D)
            rowcp = pltpu.async_copy(tab_hbm.at[r], row_v, rsem)

            @pl.when(fld != prev_fld)
            def _():
                pltpu.sync_copy(idx_hbm.at[fld], idx_v)

            prev_fld = fld
            rowcp.wait()

            for c in range(NOC):
                bsel = c % 2
                if len(waits) >= 2:
                    waits.pop(0).wait()

                @plsc.parallel_loop(0, OCH // LANES, unroll=8)
                def _(j):
                    sl = pl.ds(pl.multiple_of(c * OCH + j * LANES, 8), LANES)
                    iv = idx_v[sl]
                    vals = plsc.load_gather(row_v, [iv])
                    obuf[bsel, pl.ds(pl.multiple_of(j * LANES, 8), LANES)] = vals

                waits.append(
                    pltpu.async_copy(
                        obuf.at[bsel], out_hbm.at[r, pl.ds(c * OCH, OCH)], osem
                    )
                )
        for w in waits:
            w.wait()

    return k(idx_t, tab_t)


def kernel(indices, tables):
    idx_t = indices.T
    tab_t = jnp.transpose(tables, (0, 2, 1)).reshape(R, V)
    out_t = _sc_rowgather(idx_t, tab_t)
    return out_t.T.reshape(B, F * D)


# speedup vs baseline: 9.5315x; 1.0014x over previous
import functools

import jax
import jax.numpy as jnp
from jax import lax
from jax.experimental import pallas as pl
from jax.experimental.pallas import tpu as pltpu
from jax.experimental.pallas import tpu_sc as plsc

F = 26
V = 100000
D = 16
B = 16384

NC = 2
NS = 16
NW = NC * NS
LANES = 16

R = F * D
RPW = R // NW
OCH = 4096
NOC = B // OCH
SPIN = 400


def _sc_rowgather(idx_t, tab_t):
    mesh = plsc.VectorSubcoreMesh(core_axis_name="c", subcore_axis_name="s")

    @functools.partial(
        pl.kernel,
        mesh=mesh,
        compiler_params=pltpu.CompilerParams(
            use_tc_tiling_on_sc=True, needs_layout_passes=False
        ),
        out_type=jax.ShapeDtypeStruct((R, B), jnp.float32),
        scratch_types=[
            pltpu.VMEM((V,), jnp.float32),
            pltpu.VMEM((B,), jnp.int32),
            pltpu.VMEM((2, OCH), jnp.float32),
            pltpu.SMEM((1,), jnp.int32),
            pltpu.SemaphoreType.DMA,
            pltpu.SemaphoreType.DMA,
        ],
    )
    def k(idx_hbm, tab_hbm, out_hbm, row_v, idx_v, obuf, spin_s, osem, rsem):
        sid = lax.axis_index("s")
        wid = sid * NC + lax.axis_index("c")
        base = wid * RPW

        spin_s[0] = lax.fori_loop(
            0, sid * SPIN, lambda i, a: a + i, jnp.int32(0)
        )

        waits = []
        prev_fld = jnp.int32(-1)
        for k_row in range(RPW):
            r = base + k_row
            fld = lax.div(r, D)
            rowcp = pltpu.async_copy(tab_hbm.at[r], row_v, rsem)

            @pl.when(fld != prev_fld)
            def _():
                pltpu.sync_copy(idx_hbm.at[fld], idx_v)

            prev_fld = fld
            rowcp.wait()

            for c in range(NOC):
                bsel = c % 2
                if len(waits) >= 2:
                    waits.pop(0).wait()

                @plsc.parallel_loop(0, OCH // LANES, unroll=8)
                def _(j):
                    sl = pl.ds(pl.multiple_of(c * OCH + j * LANES, 8), LANES)
                    iv = idx_v[sl]
                    vals = plsc.load_gather(row_v, [iv])
                    obuf[bsel, pl.ds(pl.multiple_of(j * LANES, 8), LANES)] = vals

                waits.append(
                    pltpu.async_copy(
                        obuf.at[bsel], out_hbm.at[r, pl.ds(c * OCH, OCH)], osem
                    )
                )
        for w in waits:
            w.wait()

    return k(idx_t, tab_t)


def kernel(indices, tables):
    idx_t = indices.T
    tab_t = jnp.transpose(tables, (0, 2, 1)).reshape(R, V)
    out_t = _sc_rowgather(idx_t, tab_t)
    return out_t.T.reshape(B, F * D)


# speedup vs baseline: 9.5370x; 1.0006x over previous
import functools

import jax
import jax.numpy as jnp
from jax import lax
from jax.experimental import pallas as pl
from jax.experimental.pallas import tpu as pltpu
from jax.experimental.pallas import tpu_sc as plsc

F = 26
V = 100000
D = 16
B = 16384

NC = 2
NS = 16
NW = NC * NS
LANES = 16

R = F * D
RPW = R // NW
OCH = 4096
NOC = B // OCH
SPIN = 2000


def _sc_rowgather(idx_t, tab_t):
    mesh = plsc.VectorSubcoreMesh(core_axis_name="c", subcore_axis_name="s")

    @functools.partial(
        pl.kernel,
        mesh=mesh,
        compiler_params=pltpu.CompilerParams(
            use_tc_tiling_on_sc=True, needs_layout_passes=False
        ),
        out_type=jax.ShapeDtypeStruct((R, B), jnp.float32),
        scratch_types=[
            pltpu.VMEM((V,), jnp.float32),
            pltpu.VMEM((B,), jnp.int32),
            pltpu.VMEM((2, OCH), jnp.float32),
            pltpu.SMEM((1,), jnp.int32),
            pltpu.SemaphoreType.DMA,
            pltpu.SemaphoreType.DMA,
        ],
    )
    def k(idx_hbm, tab_hbm, out_hbm, row_v, idx_v, obuf, spin_s, osem, rsem):
        sid = lax.axis_index("s")
        wid = sid * NC + lax.axis_index("c")
        base = wid * RPW

        spin_s[0] = lax.fori_loop(
            0, sid * SPIN, lambda i, a: a + i, jnp.int32(0)
        )

        waits = []
        prev_fld = jnp.int32(-1)
        for k_row in range(RPW):
            r = base + k_row
            fld = lax.div(r, D)
            rowcp = pltpu.async_copy(tab_hbm.at[r], row_v, rsem)

            @pl.when(fld != prev_fld)
            def _():
                pltpu.sync_copy(idx_hbm.at[fld], idx_v)

            prev_fld = fld
            rowcp.wait()

            for c in range(NOC):
                bsel = c % 2
                if len(waits) >= 2:
                    waits.pop(0).wait()

                @plsc.parallel_loop(0, OCH // LANES, unroll=8)
                def _(j):
                    sl = pl.ds(pl.multiple_of(c * OCH + j * LANES, 8), LANES)
                    iv = idx_v[sl]
                    vals = plsc.load_gather(row_v, [iv])
                    obuf[bsel, pl.ds(pl.multiple_of(j * LANES, 8), LANES)] = vals

                waits.append(
                    pltpu.async_copy(
                        obuf.at[bsel], out_hbm.at[r, pl.ds(c * OCH, OCH)], osem
                    )
                )
        for w in waits:
            w.wait()

    return k(idx_t, tab_t)


def kernel(indices, tables):
    idx_t = indices.T
    tab_t = jnp.transpose(tables, (0, 2, 1)).reshape(R, V)
    out_t = _sc_rowgather(idx_t, tab_t)
    return out_t.T.reshape(B, F * D)
